# baseline (speedup 1.0000x reference)
import jax
import jax.numpy as jnp
from jax.experimental import pallas as pl
from jax.experimental.pallas import tpu as pltpu

_F = 64
_FD = 128
_T = 4
_C = 64
_K = 1024
_L = 4
_TH = 8
_CH = 512
_RB = 512


def _sum64(x):
    p = x[:, 0:8]
    for j in range(1, 8):
        p = p + x[:, 8 * j:8 * j + 8]
    q = p[:, 0:4] + p[:, 4:8]
    q = q[:, 0:2] + q[:, 2:4]
    return q[:, 0:1] + q[:, 1:2]


def _ln(x):
    m = _sum64(x) * (1.0 / 64.0)
    xc = x - m
    v = _sum64(xc * xc) * (1.0 / 64.0)
    return xc / jnp.sqrt(v + 1e-6)


def _mixer(z, rb, tW1_ref, tW2_ref, cW1_ref, cW2_ref, prec):
    for l in range(_L):
        y = _ln(z)
        bf = lambda a: a.astype(jnp.bfloat16).astype(jnp.float32)
        ys = [bf(y[t * rb:(t + 1) * rb]) for t in range(_T)]
        gs = []
        for h in range(_TH):
            u = ys[0] * bf(tW1_ref[l, 0, h])
            for t in range(1, _T):
                u = u + ys[t] * bf(tW1_ref[l, t, h])
            gs.append(jax.nn.gelu(u))
        gbs = [bf(g) for g in gs]
        outs = []
        for t in range(_T):
            w = gbs[0] * bf(tW2_ref[l, 0, t])
            for h in range(1, _TH):
                w = w + gbs[h] * bf(tW2_ref[l, h, t])
            outs.append(w)
        z = z + jnp.concatenate(outs, axis=0)
        y = _ln(z)
        h1 = jax.nn.gelu(jnp.dot(y, cW1_ref[l], preferred_element_type=jnp.float32,
                                 precision=prec))
        z = z + jnp.dot(h1, cW2_ref[l], preferred_element_type=jnp.float32, precision=prec)
    return z


def _body(x_ref, eW4_ref, eb_ref, etW1_ref, etW2_ref, ecW1_ref, ecW2_ref,
          cbT_ref, cb_ref, dtW1_ref, dtW2_ref, dcW1_ref, dcW2_ref,
          dW4_ref, db_ref, out_ref, idx_ref):
    rb = _RB
    frames = x_ref[...]
    df = jax.lax.Precision.DEFAULT
    hi = jax.lax.Precision.HIGHEST
    zs = [jnp.dot(frames, eW4_ref[t], preferred_element_type=jnp.float32, precision=df)
          + eb_ref[t] for t in range(_T)]
    z = jnp.concatenate(zs, axis=0)
    z = _mixer(z, rb, etW1_ref, etW2_ref, ecW1_ref, ecW2_ref, df)

    cbT = cbT_ref[...]
    cq = cbT * cbT
    p = cq[0:8]
    for j in range(1, 8):
        p = p + cq[8 * j:8 * j + 8]
    qq = p[0:4] + p[4:8]
    qq = qq[0:2] + qq[2:4]
    cbn = qq[0:1] + qq[1:2]
    zz = _sum64(z * z)
    v = jnp.dot(z, cbT, preferred_element_type=jnp.float32, precision=df)
    d = zz - 2.0 * v + cbn
    m = jnp.min(d, axis=-1, keepdims=True)
    ii = jax.lax.broadcasted_iota(jnp.int32, d.shape, 1)
    idx = jnp.min(jnp.where(d <= m, ii, _K), axis=-1, keepdims=True)
    oh = (ii == idx).astype(jnp.float32)
    q = jnp.dot(oh, cb_ref[...], preferred_element_type=jnp.float32, precision=hi)

    q = _mixer(q, rb, dtW1_ref, dtW2_ref, dcW1_ref, dcW2_ref, df)

    acc = jnp.dot(q[0:rb], dW4_ref[0], preferred_element_type=jnp.float32,
                  precision=df) + db_ref[...]
    for t in range(1, _T):
        acc = acc + jnp.dot(q[t * rb:(t + 1) * rb], dW4_ref[t],
                            preferred_element_type=jnp.float32, precision=df)
    out_ref[...] = acc
    idx_ref[...] = jnp.concatenate([idx[t * rb:(t + 1) * rb] for t in range(_T)], axis=1)


def kernel(x, enc_in_W, enc_in_b, enc_tok_W1, enc_tok_W2, enc_ch_W1, enc_ch_W2,
           codebook, dec_tok_W1, dec_tok_W2, dec_ch_W1, dec_ch_W2, dec_out_W, dec_out_b):
    Bb = x.shape[0]
    nf = Bb * _F
    xf = x.reshape(nf, _FD)
    eW4 = enc_in_W.reshape(_FD, _T, _C).transpose(1, 0, 2)
    eb = enc_in_b.reshape(_T, 1, _C)
    dW4 = dec_out_W.reshape(_T, _C, _FD)
    db = dec_out_b.reshape(1, _FD)
    cbT = codebook.T

    recon, idx = pl.pallas_call(
        _body,
        grid=(nf // _RB,),
        in_specs=[
            pl.BlockSpec((_RB, _FD), lambda i: (i, 0)),
            pl.BlockSpec((_T, _FD, _C), lambda i: (0, 0, 0)),
            pl.BlockSpec((_T, 1, _C), lambda i: (0, 0, 0)),
            pl.BlockSpec(memory_space=pltpu.SMEM),
            pl.BlockSpec(memory_space=pltpu.SMEM),
            pl.BlockSpec((_L, _C, _CH), lambda i: (0, 0, 0)),
            pl.BlockSpec((_L, _CH, _C), lambda i: (0, 0, 0)),
            pl.BlockSpec((_C, _K), lambda i: (0, 0)),
            pl.BlockSpec((_K, _C), lambda i: (0, 0)),
            pl.BlockSpec(memory_space=pltpu.SMEM),
            pl.BlockSpec(memory_space=pltpu.SMEM),
            pl.BlockSpec((_L, _C, _CH), lambda i: (0, 0, 0)),
            pl.BlockSpec((_L, _CH, _C), lambda i: (0, 0, 0)),
            pl.BlockSpec((_T, _C, _FD), lambda i: (0, 0, 0)),
            pl.BlockSpec((1, _FD), lambda i: (0, 0)),
        ],
        out_specs=[
            pl.BlockSpec((_RB, _FD), lambda i: (i, 0)),
            pl.BlockSpec((_RB, _T), lambda i: (i, 0)),
        ],
        out_shape=[
            jax.ShapeDtypeStruct((nf, _FD), jnp.float32),
            jax.ShapeDtypeStruct((nf, _T), jnp.int32),
        ],
    )(xf, eW4, eb, enc_tok_W1, enc_tok_W2, enc_ch_W1, enc_ch_W2,
      cbT, codebook, dec_tok_W1, dec_tok_W2, dec_ch_W1, dec_ch_W2, dW4, db)
    return recon.reshape(Bb, _F * _FD), idx.reshape(Bb, _F * _T)


# speedup vs baseline: 1.0784x; 1.0784x over previous
import jax
import jax.numpy as jnp
from jax.experimental import pallas as pl
from jax.experimental.pallas import tpu as pltpu

_F = 64
_FD = 128
_T = 4
_C = 64
_K = 1024
_L = 4
_TH = 8
_CH = 512
_RB = 512


def _sum64(x):
    p = x[:, 0:8]
    for j in range(1, 8):
        p = p + x[:, 8 * j:8 * j + 8]
    q = p[:, 0:4] + p[:, 4:8]
    q = q[:, 0:2] + q[:, 2:4]
    return q[:, 0:1] + q[:, 1:2]


def _ln(x):
    m = _sum64(x) * (1.0 / 64.0)
    xc = x - m
    v = _sum64(xc * xc) * (1.0 / 64.0)
    return xc / jnp.sqrt(v + 1e-6)


def _mixer(z, rb, tW1_ref, tW2_ref, cW1_ref, cW2_ref, prec):
    for l in range(_L):
        y = _ln(z)
        bf = lambda a: a.astype(jnp.bfloat16).astype(jnp.float32)
        ys = [bf(y[t * rb:(t + 1) * rb]) for t in range(_T)]
        gs = []
        for h in range(_TH):
            u = ys[0] * bf(tW1_ref[l, 0, h])
            for t in range(1, _T):
                u = u + ys[t] * bf(tW1_ref[l, t, h])
            gs.append(jax.nn.gelu(u))
        gbs = [bf(g) for g in gs]
        outs = []
        for t in range(_T):
            w = gbs[0] * bf(tW2_ref[l, 0, t])
            for h in range(1, _TH):
                w = w + gbs[h] * bf(tW2_ref[l, h, t])
            outs.append(w)
        z = z + jnp.concatenate(outs, axis=0)
        y = _ln(z)
        h1 = jax.nn.gelu(jnp.dot(y, cW1_ref[l], preferred_element_type=jnp.float32,
                                 precision=prec))
        z = z + jnp.dot(h1, cW2_ref[l], preferred_element_type=jnp.float32, precision=prec)
    return z


def _body(x_ref, eW4_ref, eb_ref, etW1_ref, etW2_ref, ecW1_ref, ecW2_ref,
          cbT_ref, cb_ref, dtW1_ref, dtW2_ref, dcW1_ref, dcW2_ref,
          dW4_ref, db_ref, out_ref, idx_ref):
    rb = _RB
    frames = x_ref[...]
    df = jax.lax.Precision.DEFAULT
    zs = [jnp.dot(frames, eW4_ref[t], preferred_element_type=jnp.float32, precision=df)
          + eb_ref[t] for t in range(_T)]
    z = jnp.concatenate(zs, axis=0)
    z = _mixer(z, rb, etW1_ref, etW2_ref, ecW1_ref, ecW2_ref, df)

    cbT = cbT_ref[...]
    cq = cbT * cbT
    p = cq[0:8]
    for j in range(1, 8):
        p = p + cq[8 * j:8 * j + 8]
    qq = p[0:4] + p[4:8]
    qq = qq[0:2] + qq[2:4]
    cbn = qq[0:1] + qq[1:2]
    zz = _sum64(z * z)
    v = jnp.dot(z, cbT, preferred_element_type=jnp.float32, precision=df)
    d = zz - 2.0 * v + cbn
    m = jnp.min(d, axis=-1, keepdims=True)
    ii = jax.lax.broadcasted_iota(jnp.int32, d.shape, 1)
    idx = jnp.min(jnp.where(d <= m, ii, _K), axis=-1, keepdims=True)
    oh = (ii == idx).astype(jnp.float32)
    cbv = cb_ref[...]
    cb_h = cbv.astype(jnp.bfloat16).astype(jnp.float32)
    cb_l = cbv - cb_h
    q = (jnp.dot(oh, cb_h, preferred_element_type=jnp.float32, precision=df)
         + jnp.dot(oh, cb_l, preferred_element_type=jnp.float32, precision=df))

    q = _mixer(q, rb, dtW1_ref, dtW2_ref, dcW1_ref, dcW2_ref, df)

    acc = jnp.dot(q[0:rb], dW4_ref[0], preferred_element_type=jnp.float32,
                  precision=df) + db_ref[...]
    for t in range(1, _T):
        acc = acc + jnp.dot(q[t * rb:(t + 1) * rb], dW4_ref[t],
                            preferred_element_type=jnp.float32, precision=df)
    out_ref[...] = acc
    idx_ref[...] = jnp.concatenate([idx[t * rb:(t + 1) * rb] for t in range(_T)], axis=1)


def kernel(x, enc_in_W, enc_in_b, enc_tok_W1, enc_tok_W2, enc_ch_W1, enc_ch_W2,
           codebook, dec_tok_W1, dec_tok_W2, dec_ch_W1, dec_ch_W2, dec_out_W, dec_out_b):
    Bb = x.shape[0]
    nf = Bb * _F
    xf = x.reshape(nf, _FD)
    eW4 = enc_in_W.reshape(_FD, _T, _C).transpose(1, 0, 2)
    eb = enc_in_b.reshape(_T, 1, _C)
    dW4 = dec_out_W.reshape(_T, _C, _FD)
    db = dec_out_b.reshape(1, _FD)
    cbT = codebook.T

    recon, idx = pl.pallas_call(
        _body,
        grid=(nf // _RB,),
        in_specs=[
            pl.BlockSpec((_RB, _FD), lambda i: (i, 0)),
            pl.BlockSpec((_T, _FD, _C), lambda i: (0, 0, 0)),
            pl.BlockSpec((_T, 1, _C), lambda i: (0, 0, 0)),
            pl.BlockSpec(memory_space=pltpu.SMEM),
            pl.BlockSpec(memory_space=pltpu.SMEM),
            pl.BlockSpec((_L, _C, _CH), lambda i: (0, 0, 0)),
            pl.BlockSpec((_L, _CH, _C), lambda i: (0, 0, 0)),
            pl.BlockSpec((_C, _K), lambda i: (0, 0)),
            pl.BlockSpec((_K, _C), lambda i: (0, 0)),
            pl.BlockSpec(memory_space=pltpu.SMEM),
            pl.BlockSpec(memory_space=pltpu.SMEM),
            pl.BlockSpec((_L, _C, _CH), lambda i: (0, 0, 0)),
            pl.BlockSpec((_L, _CH, _C), lambda i: (0, 0, 0)),
            pl.BlockSpec((_T, _C, _FD), lambda i: (0, 0, 0)),
            pl.BlockSpec((1, _FD), lambda i: (0, 0)),
        ],
        out_specs=[
            pl.BlockSpec((_RB, _FD), lambda i: (i, 0)),
            pl.BlockSpec((_RB, _T), lambda i: (i, 0)),
        ],
        out_shape=[
            jax.ShapeDtypeStruct((nf, _FD), jnp.float32),
            jax.ShapeDtypeStruct((nf, _T), jnp.int32),
        ],
    )(xf, eW4, eb, enc_tok_W1, enc_tok_W2, enc_ch_W1, enc_ch_W2,
      cbT, codebook, dec_tok_W1, dec_tok_W2, dec_ch_W1, dec_ch_W2, dW4, db)
    return recon.reshape(Bb, _F * _FD), idx.reshape(Bb, _F * _T)


# speedup vs baseline: 1.1157x; 1.0346x over previous
import jax
import jax.numpy as jnp
from jax.experimental import pallas as pl
from jax.experimental.pallas import tpu as pltpu

_F = 64
_FD = 128
_T = 4
_C = 64
_K = 1024
_L = 4
_TH = 8
_CH = 512
_RB = 1024


def _sum64(x):
    p = x[:, 0:8]
    for j in range(1, 8):
        p = p + x[:, 8 * j:8 * j + 8]
    q = p[:, 0:4] + p[:, 4:8]
    q = q[:, 0:2] + q[:, 2:4]
    return q[:, 0:1] + q[:, 1:2]


def _ln(x):
    m = _sum64(x) * (1.0 / 64.0)
    xc = x - m
    v = _sum64(xc * xc) * (1.0 / 64.0)
    return xc / jnp.sqrt(v + 1e-6)


def _mixer(z, rb, tW1_ref, tW2_ref, cW1_ref, cW2_ref, prec):
    for l in range(_L):
        y = _ln(z)
        bf = lambda a: a.astype(jnp.bfloat16).astype(jnp.float32)
        ys = [bf(y[t * rb:(t + 1) * rb]) for t in range(_T)]
        gs = []
        for h in range(_TH):
            u = ys[0] * bf(tW1_ref[l, 0, h])
            for t in range(1, _T):
                u = u + ys[t] * bf(tW1_ref[l, t, h])
            gs.append(jax.nn.gelu(u))
        gbs = [bf(g) for g in gs]
        outs = []
        for t in range(_T):
            w = gbs[0] * bf(tW2_ref[l, 0, t])
            for h in range(1, _TH):
                w = w + gbs[h] * bf(tW2_ref[l, h, t])
            outs.append(w)
        z = z + jnp.concatenate(outs, axis=0)
        y = _ln(z)
        h1 = jax.nn.gelu(jnp.dot(y, cW1_ref[l], preferred_element_type=jnp.float32,
                                 precision=prec))
        z = z + jnp.dot(h1, cW2_ref[l], preferred_element_type=jnp.float32, precision=prec)
    return z


def _body(x_ref, eW4_ref, eb_ref, etW1_ref, etW2_ref, ecW1_ref, ecW2_ref,
          cbT_ref, cb_ref, dtW1_ref, dtW2_ref, dcW1_ref, dcW2_ref,
          dW4_ref, db_ref, out_ref, idx_ref):
    rb = _RB
    frames = x_ref[...]
    df = jax.lax.Precision.DEFAULT
    zs = [jnp.dot(frames, eW4_ref[t], preferred_element_type=jnp.float32, precision=df)
          + eb_ref[t] for t in range(_T)]
    z = jnp.concatenate(zs, axis=0)
    z = _mixer(z, rb, etW1_ref, etW2_ref, ecW1_ref, ecW2_ref, df)

    cbT = cbT_ref[...]
    cq = cbT * cbT
    p = cq[0:8]
    for j in range(1, 8):
        p = p + cq[8 * j:8 * j + 8]
    qq = p[0:4] + p[4:8]
    qq = qq[0:2] + qq[2:4]
    cbn = qq[0:1] + qq[1:2]
    zz = _sum64(z * z)
    v = jnp.dot(z, cbT, preferred_element_type=jnp.float32, precision=df)
    d = zz - 2.0 * v + cbn
    m = jnp.min(d, axis=-1, keepdims=True)
    ii = jax.lax.broadcasted_iota(jnp.int32, d.shape, 1)
    idx = jnp.min(jnp.where(d <= m, ii, _K), axis=-1, keepdims=True)
    oh = (ii == idx).astype(jnp.float32)
    cbv = cb_ref[...]
    cb_h = cbv.astype(jnp.bfloat16).astype(jnp.float32)
    cb_l = cbv - cb_h
    q = (jnp.dot(oh, cb_h, preferred_element_type=jnp.float32, precision=df)
         + jnp.dot(oh, cb_l, preferred_element_type=jnp.float32, precision=df))

    q = _mixer(q, rb, dtW1_ref, dtW2_ref, dcW1_ref, dcW2_ref, df)

    acc = jnp.dot(q[0:rb], dW4_ref[0], preferred_element_type=jnp.float32,
                  precision=df) + db_ref[...]
    for t in range(1, _T):
        acc = acc + jnp.dot(q[t * rb:(t + 1) * rb], dW4_ref[t],
                            preferred_element_type=jnp.float32, precision=df)
    out_ref[...] = acc
    idx_ref[...] = jnp.concatenate([idx[t * rb:(t + 1) * rb] for t in range(_T)], axis=1)


def kernel(x, enc_in_W, enc_in_b, enc_tok_W1, enc_tok_W2, enc_ch_W1, enc_ch_W2,
           codebook, dec_tok_W1, dec_tok_W2, dec_ch_W1, dec_ch_W2, dec_out_W, dec_out_b):
    Bb = x.shape[0]
    nf = Bb * _F
    xf = x.reshape(nf, _FD)
    eW4 = enc_in_W.reshape(_FD, _T, _C).transpose(1, 0, 2)
    eb = enc_in_b.reshape(_T, 1, _C)
    dW4 = dec_out_W.reshape(_T, _C, _FD)
    db = dec_out_b.reshape(1, _FD)
    cbT = codebook.T

    recon, idx = pl.pallas_call(
        _body,
        grid=(nf // _RB,),
        in_specs=[
            pl.BlockSpec((_RB, _FD), lambda i: (i, 0)),
            pl.BlockSpec((_T, _FD, _C), lambda i: (0, 0, 0)),
            pl.BlockSpec((_T, 1, _C), lambda i: (0, 0, 0)),
            pl.BlockSpec(memory_space=pltpu.SMEM),
            pl.BlockSpec(memory_space=pltpu.SMEM),
            pl.BlockSpec((_L, _C, _CH), lambda i: (0, 0, 0)),
            pl.BlockSpec((_L, _CH, _C), lambda i: (0, 0, 0)),
            pl.BlockSpec((_C, _K), lambda i: (0, 0)),
            pl.BlockSpec((_K, _C), lambda i: (0, 0)),
            pl.BlockSpec(memory_space=pltpu.SMEM),
            pl.BlockSpec(memory_space=pltpu.SMEM),
            pl.BlockSpec((_L, _C, _CH), lambda i: (0, 0, 0)),
            pl.BlockSpec((_L, _CH, _C), lambda i: (0, 0, 0)),
            pl.BlockSpec((_T, _C, _FD), lambda i: (0, 0, 0)),
            pl.BlockSpec((1, _FD), lambda i: (0, 0)),
        ],
        out_specs=[
            pl.BlockSpec((_RB, _FD), lambda i: (i, 0)),
            pl.BlockSpec((_RB, _T), lambda i: (i, 0)),
        ],
        out_shape=[
            jax.ShapeDtypeStruct((nf, _FD), jnp.float32),
            jax.ShapeDtypeStruct((nf, _T), jnp.int32),
        ],
    )(xf, eW4, eb, enc_tok_W1, enc_tok_W2, enc_ch_W1, enc_ch_W2,
      cbT, codebook, dec_tok_W1, dec_tok_W2, dec_ch_W1, dec_ch_W2, dW4, db)
    return recon.reshape(Bb, _F * _FD), idx.reshape(Bb, _F * _T)
